# Initial kernel scaffold; baseline (speedup 1.0000x reference)
import jax
import jax.numpy as jnp
from jax.experimental import pallas as pl


def kernel(z, pos, batch, edge_index, idx_kj, idx_ji, params):
    raise NotImplementedError("write your pallas kernel here")


# speedup vs baseline: 2.1914x; 2.1914x over previous
import functools
import jax
import jax.numpy as jnp
from jax import lax
from jax.experimental import pallas as pl
from jax.experimental.pallas import tpu as pltpu
from jax.experimental.pallas import tpu_sc as plsc

H = 64
OUT_EMB = 256
NSPH = 7
NRAD = 6
CUTOFF = 5.0
NATOM = 95
N = 10000
E = 160000
T = 480000
G = 512

RE = 640
RT = 640
RN = 512
NPAD = 10240


def _sc_gather(table, idx, ch):
    B = idx.shape[0]
    D = table.shape[1]
    info = plsc.get_sparse_core_info()
    nw = info.num_cores * info.num_subcores
    bpw = B // nw
    nch = bpw // ch
    mesh = plsc.VectorSubcoreMesh(core_axis_name="c", subcore_axis_name="s")

    @functools.partial(
        pl.kernel,
        mesh=mesh,
        compiler_params=pltpu.CompilerParams(use_tc_tiling_on_sc=False),
        out_type=jax.ShapeDtypeStruct((B, D), jnp.float32),
        scratch_types=[
            pltpu.VMEM((ch,), jnp.int32),
            pltpu.VMEM((ch, D), jnp.float32),
            pltpu.SemaphoreType.DMA,
        ],
    )
    def k(table_hbm, idx_hbm, out_hbm, idx_v, rows_v, sem):
        wid = lax.axis_index("s") * info.num_cores + lax.axis_index("c")
        base = wid * bpw

        def body(c, carry):
            off = base + c * ch
            pltpu.sync_copy(idx_hbm.at[pl.ds(off, ch)], idx_v)
            pltpu.async_copy(table_hbm.at[idx_v], rows_v, sem).wait()
            pltpu.sync_copy(rows_v, out_hbm.at[pl.ds(off, ch)])
            return carry

        lax.fori_loop(0, nch, body, 0)

    return k(table, idx)


def _silu(v):
    return v * jax.nn.sigmoid(v)


def _row_spec(r, c):
    return pl.BlockSpec((r, c), lambda i: (i, 0))


def _full_spec(shape):
    return pl.BlockSpec(shape, lambda i: (0, 0))


def _pad_rows(a, rows):
    return jnp.pad(a, ((0, rows - a.shape[0]), (0, 0)))


def _bias8(b):
    return jnp.pad(b[None, :], ((0, 7), (0, 0)))


def _mmb(x_ref, w_ref, b_ref):
    return jnp.dot(x_ref[...], w_ref[...],
                   preferred_element_type=jnp.float32) + b_ref[0:1, :]


def _edge_kernel(pj_ref, pi_ref, emb96_ref, erbf_w_ref, erbf_b_ref,
                 w1_ref, w2_ref, w3_ref, lb_ref,
                 geom_ref, x_ref, rbf8_ref):
    pj = pj_ref[...]
    pi = pi_ref[...]
    diff = pi[:, 0:3] - pj[:, 0:3]
    d = jnp.sqrt(jnp.sum(diff * diff, axis=-1, keepdims=True) + 1e-6)
    geom_ref[...] = jnp.concatenate(
        [diff, d, jnp.zeros((diff.shape[0], 12), jnp.float32)], axis=1)

    xx = d * (1.0 / CUTOFF)
    p = 6.0
    a = -(p + 1.0) * (p + 2.0) / 2.0
    bcoef = p * (p + 2.0)
    cc = -p * (p + 1.0) / 2.0
    x2 = xx * xx
    x4 = x2 * x2
    x5 = x4 * xx
    env = (1.0 / xx + a * x5 + bcoef * x5 * xx + cc * x5 * x2) * (xx < 1.0)
    krow = lax.broadcasted_iota(jnp.int32, (1, 8), 1).astype(jnp.float32) + 1.0
    rbf8 = env * jnp.sin(krow * jnp.pi * xx)
    rbf8_ref[...] = rbf8

    rbf_h = _silu(jnp.dot(rbf8, erbf_w_ref[...],
                          preferred_element_type=jnp.float32) + erbf_b_ref[0:1, :])

    ar = lax.broadcasted_iota(jnp.int32, (pj.shape[0], 96), 1).astype(jnp.float32)
    oh_j = (ar == pj[:, 3:4]).astype(jnp.float32)
    oh_i = (ar == pi[:, 3:4]).astype(jnp.float32)
    xj = jnp.dot(oh_j, emb96_ref[...], preferred_element_type=jnp.float32)
    xi = jnp.dot(oh_i, emb96_ref[...], preferred_element_type=jnp.float32)
    x_ref[...] = _silu(
        jnp.dot(xj, w1_ref[...], preferred_element_type=jnp.float32)
        + jnp.dot(xi, w2_ref[...], preferred_element_type=jnp.float32)
        + jnp.dot(rbf_h, w3_ref[...], preferred_element_type=jnp.float32)
        + lb_ref[0:1, :])


def _sbf_kernel(v1_ref, v2_ref, s1m0_ref, s1m1_ref, s20_ref, s21_ref,
                o0_ref, o1_ref):
    v1 = v1_ref[...][:, 0:3]
    v2g = v2_ref[...]
    v2 = v2g[:, 0:3]
    dkj = v2g[:, 3:4]
    dot = jnp.sum(v1 * v2, axis=-1, keepdims=True)
    cx = v1[:, 1:2] * v2[:, 2:3] - v1[:, 2:3] * v2[:, 1:2]
    cy = v1[:, 2:3] * v2[:, 0:1] - v1[:, 0:1] * v2[:, 2:3]
    cz = v1[:, 0:1] * v2[:, 1:2] - v1[:, 1:2] * v2[:, 0:1]
    cn2 = cx * cx + cy * cy + cz * cz + 1e-9
    c = dot * lax.rsqrt(dot * dot + cn2)

    xx = dkj * (1.0 / CUTOFF)
    p = 6.0
    a = -(p + 1.0) * (p + 2.0) / 2.0
    bcoef = p * (p + 2.0)
    cc = -p * (p + 1.0) / 2.0
    x2 = xx * xx
    x4 = x2 * x2
    x5 = x4 * xx
    env = (1.0 / xx + a * x5 + bcoef * x5 * xx + cc * x5 * x2) * (xx < 1.0)
    krow = lax.broadcasted_iota(jnp.int32, (1, 8), 1).astype(jnp.float32) + 1.0
    rad8 = env * jnp.sin(krow * jnp.pi * xx)

    angs = [jnp.ones_like(c), c]
    for _ in range(5):
        angs.append(2.0 * c * angs[-1] - angs[-2])
    sbf56 = jnp.concatenate([angs[l] * rad8 for l in range(7)], axis=1)
    t80 = jnp.dot(sbf56, s1m0_ref[...], preferred_element_type=jnp.float32)
    t81 = jnp.dot(sbf56, s1m1_ref[...], preferred_element_type=jnp.float32)
    o0_ref[...] = jnp.dot(t80, s20_ref[...], preferred_element_type=jnp.float32)
    o1_ref[...] = jnp.dot(t81, s21_ref[...], preferred_element_type=jnp.float32)


def _pre_kernel(x_ref, rbf8_ref, jiw_ref, jib_ref, kjw_ref, kjb_ref,
                r1_ref, r2_ref, dn_ref, xji_ref, xdn_ref):
    x = x_ref[...]
    rbf8 = rbf8_ref[...]
    xji_ref[...] = _silu(_mmb(x_ref, jiw_ref, jib_ref))
    xkj = _silu(_mmb(x_ref, kjw_ref, kjb_ref))
    g = jnp.dot(jnp.dot(rbf8, r1_ref[...], preferred_element_type=jnp.float32),
                r2_ref[...], preferred_element_type=jnp.float32)
    xkj = xkj * g
    xdn_ref[...] = _silu(jnp.dot(xkj, dn_ref[...],
                                 preferred_element_type=jnp.float32))


def _mul_kernel(a_ref, b_ref, o_ref):
    o_ref[...] = a_ref[...] * b_ref[...]


def _post_kernel(seg_ref, xji_ref, x_ref, up_ref,
                 bw1_ref, bb1_ref, bw2_ref, bb2_ref,
                 lw_ref, lb_ref,
                 a1w1_ref, a1b1_ref, a1w2_ref, a1b2_ref,
                 a2w1_ref, a2b1_ref, a2w2_ref, a2b2_ref,
                 out_ref):
    xkj = _silu(jnp.dot(seg_ref[...], up_ref[...],
                        preferred_element_type=jnp.float32))
    h = xji_ref[...] + xkj
    h = h + _silu(jnp.dot(_silu(_mmb(h, bw1_ref, bb1_ref)), bw2_ref[...],
                          preferred_element_type=jnp.float32) + bb2_ref[0:1, :])
    h = _silu(jnp.dot(h, lw_ref[...],
                      preferred_element_type=jnp.float32) + lb_ref[0:1, :]) \
        + x_ref[...]
    h = h + _silu(jnp.dot(_silu(_mmb(h, a1w1_ref, a1b1_ref)), a1w2_ref[...],
                          preferred_element_type=jnp.float32) + a1b2_ref[0:1, :])
    h = h + _silu(jnp.dot(_silu(_mmb(h, a2w1_ref, a2b1_ref)), a2w2_ref[...],
                          preferred_element_type=jnp.float32) + a2b2_ref[0:1, :])
    out_ref[...] = h


def _gx_kernel(x_ref, rbf8_ref, rw_ref, o_ref):
    o_ref[...] = x_ref[...] * jnp.dot(rbf8_ref[...], rw_ref[...],
                                      preferred_element_type=jnp.float32)


def _node_kernel(nd_ref, up_ref, w1_ref, b1_ref, w2_ref, b2_ref,
                 w3_ref, b3_ref, ow_ref, o_ref):
    h = jnp.dot(nd_ref[...], up_ref[...], preferred_element_type=jnp.float32)
    h = _silu(_mmb(h, w1_ref, b1_ref))
    h = _silu(_mmb(h, w2_ref, b2_ref))
    h = _silu(_mmb(h, w3_ref, b3_ref))
    o_ref[...] = jnp.dot(h, ow_ref[...], preferred_element_type=jnp.float32)


def _call_rows(body, nrows, rblk, outs, ins, in_specs, out_specs):
    return pl.pallas_call(
        body,
        grid=(nrows // rblk,),
        in_specs=in_specs,
        out_specs=out_specs,
        out_shape=outs,
    )(*ins)


def _sbf1_mod(s1):
    m = jnp.zeros((56, 8), jnp.float32)
    return m.at[
        (jnp.arange(NSPH)[:, None] * 8 + jnp.arange(NRAD)[None, :]).reshape(-1)
    ].set(s1.reshape(42, 8))


def kernel(z, pos, batch, edge_index, idx_kj, idx_ji, params):
    f32 = jnp.float32
    z = z.astype(jnp.int32)
    j = edge_index[0].astype(jnp.int32)
    i = edge_index[1].astype(jnp.int32)
    idx_kj = idx_kj.astype(jnp.int32)
    idx_ji = idx_ji.astype(jnp.int32)

    ptab = jnp.concatenate(
        [pos.astype(f32), z.astype(f32)[:, None], jnp.zeros((N, 12), f32)],
        axis=1)
    pj = _sc_gather(ptab, j, 1000)
    pi = _sc_gather(ptab, i, 1000)

    emb96 = _pad_rows(params['emb'], 96)
    erbf_w = _pad_rows(params['emb_rbf_w'], 8)
    lw = params['emb_lin_w']
    w1, w2, w3 = lw[0:H], lw[H:2 * H], lw[2 * H:3 * H]

    geom, x, rbf8 = _call_rows(
        _edge_kernel, E, RE,
        (jax.ShapeDtypeStruct((E, 16), f32),
         jax.ShapeDtypeStruct((E, H), f32),
         jax.ShapeDtypeStruct((E, 8), f32)),
        (pj, pi, emb96, erbf_w, _bias8(params['emb_rbf_b']),
         w1, w2, w3, _bias8(params['emb_lin_b'])),
        [_row_spec(RE, 16), _row_spec(RE, 16), _full_spec((96, H)),
         _full_spec((8, H)), _full_spec((8, H)),
         _full_spec((H, H)), _full_spec((H, H)), _full_spec((H, H)),
         _full_spec((8, H))],
        [_row_spec(RE, 16), _row_spec(RE, H), _row_spec(RE, 8)])

    v1 = _sc_gather(geom, idx_ji, 1000)
    v2 = _sc_gather(geom, idx_kj, 1000)

    blks = params['blocks']
    sbt0, sbt1 = _call_rows(
        _sbf_kernel, T, RT,
        (jax.ShapeDtypeStruct((T, H), f32), jax.ShapeDtypeStruct((T, H), f32)),
        (v1, v2, _sbf1_mod(blks[0]['sbf1']), _sbf1_mod(blks[1]['sbf1']),
         blks[0]['sbf2'], blks[1]['sbf2']),
        [_row_spec(RT, 16), _row_spec(RT, 16), _full_spec((56, 8)),
         _full_spec((56, 8)), _full_spec((8, H)), _full_spec((8, H))],
        [_row_spec(RT, H), _row_spec(RT, H)])

    obs = params['out_blocks']

    def out_block(ob, x_e):
        gx = _call_rows(
            _gx_kernel, E, RE, jax.ShapeDtypeStruct((E, H), f32),
            (x_e, rbf8, _pad_rows(ob['rbf_w'], 8)),
            [_row_spec(RE, H), _row_spec(RE, 8), _full_spec((8, H))],
            _row_spec(RE, H))
        node = jax.ops.segment_sum(gx, i, num_segments=N)
        node = _pad_rows(node, NPAD)
        (l1w, l1b), (l2w, l2b), (l3w, l3b) = ob['lins']
        pv = _call_rows(
            _node_kernel, NPAD, RN, jax.ShapeDtypeStruct((NPAD, 128), f32),
            (node, ob['up'], l1w, _bias8(l1b), l2w, _bias8(l2b),
             l3w, _bias8(l3b), jnp.pad(ob['out'], ((0, 0), (0, 127)))),
            [_row_spec(RN, H), _full_spec((H, OUT_EMB)),
             _full_spec((OUT_EMB, OUT_EMB)), _full_spec((8, OUT_EMB)),
             _full_spec((OUT_EMB, OUT_EMB)), _full_spec((8, OUT_EMB)),
             _full_spec((OUT_EMB, OUT_EMB)), _full_spec((8, OUT_EMB)),
             _full_spec((OUT_EMB, 128))],
            _row_spec(RN, 128))
        return pv[:N, 0]

    P = out_block(obs[0], x)

    for bi, blk in enumerate(blks):
        sbt = sbt0 if bi == 0 else sbt1
        xji, xdn = _call_rows(
            _pre_kernel, E, RE,
            (jax.ShapeDtypeStruct((E, H), f32),
             jax.ShapeDtypeStruct((E, H), f32)),
            (x, rbf8, blk['ji_w'], _bias8(blk['ji_b']),
             blk['kj_w'], _bias8(blk['kj_b']),
             jnp.pad(blk['rbf1'], ((0, 2), (0, 0))), blk['rbf2'], blk['down']),
            [_row_spec(RE, H), _row_spec(RE, 8), _full_spec((H, H)),
             _full_spec((8, H)), _full_spec((H, H)), _full_spec((8, H)),
             _full_spec((8, 8)), _full_spec((8, H)), _full_spec((H, H))],
            [_row_spec(RE, H), _row_spec(RE, H)])

        gkj = _sc_gather(xdn, idx_kj, 1000)
        m = _call_rows(
            _mul_kernel, T, RT, jax.ShapeDtypeStruct((T, H), f32),
            (gkj, sbt),
            [_row_spec(RT, H), _row_spec(RT, H)],
            _row_spec(RT, H))
        seg = jax.ops.segment_sum(m, idx_ji, num_segments=E)

        x = _call_rows(
            _post_kernel, E, RE, jax.ShapeDtypeStruct((E, H), f32),
            (seg, xji, x, blk['up'],
             blk['before'][0][0], _bias8(blk['before'][0][1]),
             blk['before'][0][2], _bias8(blk['before'][0][3]),
             blk['lin_w'], _bias8(blk['lin_b']),
             blk['after'][0][0], _bias8(blk['after'][0][1]),
             blk['after'][0][2], _bias8(blk['after'][0][3]),
             blk['after'][1][0], _bias8(blk['after'][1][1]),
             blk['after'][1][2], _bias8(blk['after'][1][3])),
            [_row_spec(RE, H)] * 3 + [_full_spec((H, H))]
            + [_full_spec((H, H)), _full_spec((8, H))] * 7,
            _row_spec(RE, H))

        P = P + out_block(obs[bi + 1], x)

    g_out = jax.ops.segment_sum(P, batch, num_segments=G)
    return g_out.reshape(-1)
